# baseline (speedup 1.0000x reference)
import functools

import jax
import jax.numpy as jnp
from jax.experimental import pallas as pl
from jax.experimental.pallas import tpu as pltpu


def _body(adj_ref, feat_ref, w_ref, b_ref, out_ref):
    adj = adj_ref[...]
    f = feat_ref[...]
    mask = jnp.max(adj, axis=-1) >= 0
    t = jnp.dot(f, w_ref[...], preferred_element_type=jnp.float32)
    t = jnp.maximum(t + b_ref[...], 0.0)
    out_ref[...] = jnp.where(mask[:, None], t, f)


@jax.jit
def kernel(adjacency, features, kernel, bias):
    B, V, R, NB = adjacency.shape
    F = features.shape[-1]
    U = kernel.shape[-1]
    N = B * V
    adj2 = adjacency.reshape(N, R * NB)
    feat2 = features.reshape(N, F)
    rows = 2000
    grid = (N // rows,)
    out = pl.pallas_call(
        _body,
        grid=grid,
        in_specs=[
            pl.BlockSpec((rows, R * NB), lambda i: (i, 0)),
            pl.BlockSpec((rows, F), lambda i: (i, 0)),
            pl.BlockSpec((F, U), lambda i: (0, 0)),
            pl.BlockSpec((1, U), lambda i: (0, 0)),
        ],
        out_specs=pl.BlockSpec((rows, U), lambda i: (i, 0)),
        out_shape=jax.ShapeDtypeStruct((N, U), jnp.float32),
    )(adj2, feat2, kernel, bias.reshape(1, U))
    return out.reshape(B, V, U)


# speedup vs baseline: 1.6940x; 1.6940x over previous
import jax
import jax.numpy as jnp
from jax.experimental import pallas as pl
from jax.experimental.pallas import tpu as pltpu


def _body(adj_ref, feat_ref, w_ref, b_ref, out_ref):
    adj = adj_ref[0]
    f = feat_ref[0]
    ind = jnp.where(adj >= 0, 1.0, 0.0)
    cnt = jnp.dot(ind, jnp.ones((ind.shape[1], f.shape[1]), jnp.float32),
                  preferred_element_type=jnp.float32)
    t = jnp.dot(f, w_ref[...], preferred_element_type=jnp.float32)
    t = jnp.maximum(t + b_ref[...], 0.0)
    out_ref[0] = jnp.where(cnt > 0.0, t, f)


@jax.jit
def kernel(adjacency, features, kernel, bias):
    B, V, R, NB = adjacency.shape
    F = features.shape[-1]
    U = kernel.shape[-1]
    adj3 = adjacency.reshape(B, V, R * NB)
    rows = 2000
    grid = (B, V // rows)
    out = pl.pallas_call(
        _body,
        grid=grid,
        in_specs=[
            pl.BlockSpec((1, rows, R * NB), lambda b, i: (b, i, 0)),
            pl.BlockSpec((1, rows, F), lambda b, i: (b, i, 0)),
            pl.BlockSpec((F, U), lambda b, i: (0, 0)),
            pl.BlockSpec((1, U), lambda b, i: (0, 0)),
        ],
        out_specs=pl.BlockSpec((1, rows, U), lambda b, i: (b, i, 0)),
        out_shape=jax.ShapeDtypeStruct((B, V, U), jnp.float32),
    )(adj3, features, kernel, bias.reshape(1, U))
    return out
